# baseline (speedup 1.0000x reference)
import jax
import jax.numpy as jnp
from jax.experimental import pallas as pl
from jax.experimental.pallas import tpu as pltpu


def _copy_body(x_ref, o_ref):
    o_ref[...] = x_ref[...]


def kernel(x, w1, w2):
    B, C, H, W = x.shape
    HW = H * W
    x2 = x.reshape(B, C * HW)
    bt = 8
    steps = B // bt
    out2 = pl.pallas_call(
        _copy_body,
        out_shape=jax.ShapeDtypeStruct((B, C * HW), x.dtype),
        grid=(steps,),
        in_specs=[
            pl.BlockSpec((bt, C * HW), lambda b: (b, 0)),
        ],
        out_specs=pl.BlockSpec((bt, C * HW), lambda b: (b, 0)),
        compiler_params=pltpu.CompilerParams(
            dimension_semantics=("parallel",),
            vmem_limit_bytes=56 * 1024 * 1024,
        ),
    )(x2)
    return out2.reshape(B, C, H, W)


# speedup vs baseline: 9.3608x; 9.3608x over previous
import functools

import jax
import jax.numpy as jnp
from jax import lax
from jax.experimental import pallas as pl
from jax.experimental.pallas import tpu as pltpu

_VMEM_LIMIT_BYTES = 56 * 1024 * 1024
_SLAB_TARGET_BYTES = 7 * 1024 * 1024


def _se_body(x_ref, w1_ref, w2_ref, o_ref, *, inv_hw, split):
    x = x_ref[...]
    hw = x.shape[0]
    if hw % split == 0 and hw > split:
        xr = x.reshape(hw // split, split, x.shape[1], x.shape[2])
        partial = jnp.sum(xr.astype(jnp.float32), axis=1)
        pooled = jnp.sum(partial, axis=0) * inv_hw
    else:
        pooled = jnp.sum(x.astype(jnp.float32), axis=0) * inv_hw
    h = lax.dot_general(pooled, w1_ref[...],
                        dimension_numbers=(((1,), (1,)), ((), ())),
                        preferred_element_type=jnp.float32)
    h = jnp.maximum(h, 0.0)
    z = lax.dot_general(h, w2_ref[...],
                        dimension_numbers=(((1,), (1,)), ((), ())),
                        preferred_element_type=jnp.float32)
    s = jax.nn.sigmoid(z).astype(x.dtype)
    o_ref[...] = (x * s[None, :, :]).astype(o_ref.dtype)


def _pick_batch_tile(B, slab_bytes_per_b):
    divisors = [d for d in range(8, B + 1, 8) if B % d == 0]
    fitting = [d for d in divisors if d * slab_bytes_per_b <= _SLAB_TARGET_BYTES]
    if not fitting:
        return 8 if B % 8 == 0 else 1
    even_steps = [d for d in fitting if (B // d) % 2 == 0]
    pool = even_steps if even_steps else fitting
    return max(pool)


def kernel(x, w1, w2):
    B, C, H, W = x.shape
    HW = H * W
    Cr = w1.shape[0]
    elt = jnp.dtype(x.dtype).itemsize

    xt = jnp.transpose(x, (2, 3, 0, 1)).reshape(HW, B, C)

    bt = _pick_batch_tile(B, C * HW * elt)
    steps = B // bt

    body = functools.partial(_se_body, inv_hw=float(1.0 / HW), split=16)
    out_t = pl.pallas_call(
        body,
        out_shape=jax.ShapeDtypeStruct((HW, B, C), x.dtype),
        grid=(steps,),
        in_specs=[
            pl.BlockSpec((HW, bt, C), lambda b: (0, b, 0)),
            pl.BlockSpec((Cr, C), lambda b: (0, 0)),
            pl.BlockSpec((C, Cr), lambda b: (0, 0)),
        ],
        out_specs=pl.BlockSpec((HW, bt, C), lambda b: (0, b, 0)),
        compiler_params=pltpu.CompilerParams(
            dimension_semantics=("parallel",),
            vmem_limit_bytes=_VMEM_LIMIT_BYTES,
        ),
        cost_estimate=pl.CostEstimate(
            flops=2 * B * C * HW + 4 * B * C * Cr,
            transcendentals=B * C,
            bytes_accessed=2 * B * C * HW * elt + 2 * C * Cr * 4,
        ),
    )(xt, w1, w2)

    return jnp.transpose(out_t.reshape(H, W, B, C), (2, 3, 0, 1))


# speedup vs baseline: 9.6508x; 1.0310x over previous
import functools

import jax
import jax.numpy as jnp
from jax import lax
from jax.experimental import pallas as pl
from jax.experimental.pallas import tpu as pltpu

_VMEM_LIMIT_BYTES = 56 * 1024 * 1024
_SLAB_TARGET_BYTES = 13 * 1024 * 1024


def _se_body(x_ref, w1_ref, w2_ref, o_ref, *, inv_hw, split):
    x = x_ref[...]
    hw = x.shape[0]
    if hw % split == 0 and hw > split:
        xr = x.reshape(hw // split, split, x.shape[1], x.shape[2])
        partial = jnp.sum(xr.astype(jnp.float32), axis=1)
        pooled = jnp.sum(partial, axis=0) * inv_hw
    else:
        pooled = jnp.sum(x.astype(jnp.float32), axis=0) * inv_hw
    h = lax.dot_general(pooled, w1_ref[...],
                        dimension_numbers=(((1,), (1,)), ((), ())),
                        preferred_element_type=jnp.float32)
    h = jnp.maximum(h, 0.0)
    z = lax.dot_general(h, w2_ref[...],
                        dimension_numbers=(((1,), (1,)), ((), ())),
                        preferred_element_type=jnp.float32)
    s = jax.nn.sigmoid(z).astype(x.dtype)
    o_ref[...] = (x * s[None, :, :]).astype(o_ref.dtype)


def _pick_batch_tile(B, slab_bytes_per_b):
    divisors = [d for d in range(8, B + 1, 8) if B % d == 0]
    fitting = [d for d in divisors if d * slab_bytes_per_b <= _SLAB_TARGET_BYTES]
    if not fitting:
        return 8 if B % 8 == 0 else 1
    even_steps = [d for d in fitting if (B // d) % 2 == 0]
    pool = even_steps if even_steps else fitting
    return max(pool)


def kernel(x, w1, w2):
    B, C, H, W = x.shape
    HW = H * W
    Cr = w1.shape[0]
    elt = jnp.dtype(x.dtype).itemsize

    xt = jnp.transpose(x, (2, 3, 0, 1)).reshape(HW, B, C)

    bt = _pick_batch_tile(B, C * HW * elt)
    steps = B // bt

    body = functools.partial(_se_body, inv_hw=float(1.0 / HW), split=16)
    out_t = pl.pallas_call(
        body,
        out_shape=jax.ShapeDtypeStruct((HW, B, C), x.dtype),
        grid=(steps,),
        in_specs=[
            pl.BlockSpec((HW, bt, C), lambda b: (0, b, 0)),
            pl.BlockSpec((Cr, C), lambda b: (0, 0)),
            pl.BlockSpec((C, Cr), lambda b: (0, 0)),
        ],
        out_specs=pl.BlockSpec((HW, bt, C), lambda b: (0, b, 0)),
        compiler_params=pltpu.CompilerParams(
            dimension_semantics=("parallel",),
            vmem_limit_bytes=_VMEM_LIMIT_BYTES,
        ),
        cost_estimate=pl.CostEstimate(
            flops=2 * B * C * HW + 4 * B * C * Cr,
            transcendentals=B * C,
            bytes_accessed=2 * B * C * HW * elt + 2 * C * Cr * 4,
        ),
    )(xt, w1, w2)

    return jnp.transpose(out_t.reshape(H, W, B, C), (2, 3, 0, 1))
